# baseline (speedup 1.0000x reference)
import functools

import jax
import jax.numpy as jnp
from jax import lax
from jax.experimental import pallas as pl
from jax.experimental.pallas import tpu as pltpu
from jax.experimental.pallas import tpu_sc as plsc

B, FN, FC, V, D, NCLS = 16384, 13, 26, 16, 16, 2
NC, NS, LANES = 2, 16, 16
NW = NC * NS
CH = B // NW
CHK = 128
NCHK = CH // CHK
NBLK = CHK // LANES

_TAB_OFF = 0
_WEMB_OFF = _TAB_OFF + FC * D * V
CONST_LEN = _WEMB_OFF + NCLS * FC * D


def _sc_body(consts_hbm, xcat_hbm, out0_hbm, out1_hbm,
             consts_v, l_v, o0_v, o1_v, xcat0, xcat1,
             csem, isem0, isem1, osem):
    cid = lax.axis_index("c")
    sid = lax.axis_index("s")
    wid = sid * NC + cid
    base = wid * CH

    xcats = [xcat0, xcat1]
    isems = [isem0, isem1]

    cp_con = pltpu.async_copy(consts_hbm, consts_v, csem)

    def start_in(k):
        s = k & 1
        return pltpu.async_copy(
            xcat_hbm.at[pl.ds(base + k * CHK, CHK), :], xcats[s], isems[s])

    pend = {0: start_in(0)}

    cp_con.wait()

    for i in range(FC):
        for c in range(NCLS):
            wvec = consts_v[pl.ds(_WEMB_OFF + (c * FC + i) * D, LANES)]
            acc = jnp.zeros((LANES,), jnp.float32)
            for d in range(D):
                acc = acc + consts_v[pl.ds(_TAB_OFF + i * (D * V) + d * V,
                                           LANES)] * wvec[d]
            l_v[pl.ds(c * (FC * V) + i * V, LANES)] = acc

    iot = lax.iota(jnp.int32, LANES)
    for k in range(NCHK):
        s = k & 1
        if k + 1 < NCHK:
            pend[k + 1] = start_in(k + 1)
        pend.pop(k).wait()
        xcat_v = xcats[s]
        koff = k * CHK

        def blk(j, carry):
            rb = j * LANES
            rows = rb + iot
            acc0 = jnp.zeros((LANES,), jnp.float32)
            acc1 = jnp.zeros((LANES,), jnp.float32)
            for i in range(FC):
                col = jnp.full((LANES,), i, jnp.int32)
                ci = plsc.load_gather(xcat_v, [rows, col])
                acc0 = acc0 + plsc.load_gather(l_v, [ci + i * V])
                acc1 = acc1 + plsc.load_gather(l_v, [ci + (FC + i) * V])
            o0_v[pl.ds(koff + rb, LANES)] = acc0
            o1_v[pl.ds(koff + rb, LANES)] = acc1
            return carry

        lax.fori_loop(0, NBLK, blk, 0)

    cp0 = pltpu.async_copy(o0_v, out0_hbm.at[pl.ds(base, CH)], osem)
    cp1 = pltpu.async_copy(o1_v, out1_hbm.at[pl.ds(base, CH)], osem)
    cp0.wait()
    cp1.wait()


def _tc_body(xnum_ref, o0_ref, o1_ref, wnt_ref, b_ref, out_ref):
    dense = jnp.dot(xnum_ref[...], wnt_ref[...],
                    preferred_element_type=jnp.float32)
    cat = jnp.concatenate([o0_ref[...][:, None], o1_ref[...][:, None]],
                          axis=1)
    out_ref[...] = dense + cat + b_ref[...]


def kernel(x_num, x_cat, tables, W, b):
    x_cat_i = x_cat.astype(jnp.int32)
    tabt = tables.transpose(0, 2, 1).reshape(-1)
    wemb = W[:, FN:].reshape(-1)
    consts = jnp.concatenate([tabt, wemb])
    wnt = W[:, :FN].T
    b2 = b[None, :]

    mesh = plsc.VectorSubcoreMesh(core_axis_name="c", subcore_axis_name="s")
    run = functools.partial(
        pl.kernel,
        mesh=mesh,
        compiler_params=pltpu.CompilerParams(needs_layout_passes=False,
                                             skip_device_barrier=True,
                                             use_tc_tiling_on_sc=True),
        out_type=(jax.ShapeDtypeStruct((B,), jnp.float32),
                  jax.ShapeDtypeStruct((B,), jnp.float32)),
        scratch_types=[
            pltpu.VMEM((CONST_LEN,), jnp.float32),
            pltpu.VMEM((NCLS * FC * V,), jnp.float32),
            pltpu.VMEM((CH,), jnp.float32),
            pltpu.VMEM((CH,), jnp.float32),
            pltpu.VMEM((CHK, FC), jnp.int32),
            pltpu.VMEM((CHK, FC), jnp.int32),
            pltpu.SemaphoreType.DMA,
            pltpu.SemaphoreType.DMA,
            pltpu.SemaphoreType.DMA,
            pltpu.SemaphoreType.DMA,
        ],
    )(_sc_body)
    out0, out1 = run(consts, x_cat_i)

    grid = 8
    rows = B // grid
    out = pl.pallas_call(
        _tc_body,
        grid=(grid,),
        in_specs=[
            pl.BlockSpec((rows, FN), lambda j: (j, 0)),
            pl.BlockSpec((rows,), lambda j: (j,)),
            pl.BlockSpec((rows,), lambda j: (j,)),
            pl.BlockSpec((FN, NCLS), lambda j: (0, 0)),
            pl.BlockSpec((1, NCLS), lambda j: (0, 0)),
        ],
        out_specs=pl.BlockSpec((rows, NCLS), lambda j: (j, 0)),
        out_shape=jax.ShapeDtypeStruct((B, NCLS), jnp.float32),
    )(x_num, out0, out1, wnt, b2)
    return out
